# Initial kernel scaffold; baseline (speedup 1.0000x reference)
import jax
import jax.numpy as jnp
from jax.experimental import pallas as pl


def kernel(feat0, feat1, feat2, feat3, rois):
    raise NotImplementedError("write your pallas kernel here")


# speedup vs baseline: 18.9840x; 18.9840x over previous
import functools

import jax
import jax.numpy as jnp
from jax import lax
from jax.experimental import pallas as pl
from jax.experimental.pallas import tpu as pltpu
from jax.experimental.pallas import tpu_sc as plsc

PH, PW = 7, 7
R = 5000
NB = PH * PW
NE = NB * 4
C = 256
RB = 1000

LEVEL_W = (256, 128, 64, 32)
LEVEL_BASE = (0, 65536, 81920, 86016)
TABLE_ROWS = 87040

NW = 32
CH4 = 128
M4 = R * NE
PER_W4 = -(-M4 // (NW * CH4)) * CH4
CHUNKS = PER_W4 // CH4
M4_PAD = PER_W4 * NW
OUT_ROWS_PAD = M4_PAD // 4


def _tc_index_kernel(rois_ref, idx_ref, w_ref):
    rois = rois_ref[...]
    x1 = rois[:, 1:2]
    y1 = rois[:, 2:3]
    x2 = rois[:, 3:4]
    y2 = rois[:, 4:5]
    bw = x2 - x1 + 1.0
    bh = y2 - y1 + 1.0
    fid = jnp.clip(
        jnp.floor(2.0 + jnp.log2(jnp.sqrt(bw * bh) / 224.0 + 1e-6)), 0.0, 3.0
    ).astype(jnp.int32)
    scale = jnp.where(
        fid == 0, 0.25, jnp.where(fid == 1, 0.125, jnp.where(fid == 2, 0.0625, 0.03125))
    ).astype(jnp.float32)
    wl = jnp.where(fid == 0, LEVEL_W[0],
                   jnp.where(fid == 1, LEVEL_W[1],
                             jnp.where(fid == 2, LEVEL_W[2], LEVEL_W[3])))
    basel = jnp.where(fid == 0, LEVEL_BASE[0],
                      jnp.where(fid == 1, LEVEL_BASE[1],
                                jnp.where(fid == 2, LEVEL_BASE[2], LEVEL_BASE[3])))

    lane = lax.broadcasted_iota(jnp.int32, (RB, NE), 1)
    k = lane // 4
    corner = lane - 4 * k
    bi = k // PW
    bj = k - PW * bi
    dy = corner // 2
    dx = corner - 2 * dy

    x1s = x1 * scale
    y1s = y1 * scale
    roi_w = jnp.maximum(x2 * scale - x1s, 1.0)
    roi_h = jnp.maximum(y2 * scale - y1s, 1.0)
    bin_w = roi_w / PW
    bin_h = roi_h / PH
    px = x1s + (bj.astype(jnp.float32) + 0.5) * bin_w
    py = y1s + (bi.astype(jnp.float32) + 0.5) * bin_h
    x0f = jnp.floor(px)
    y0f = jnp.floor(py)
    lx = px - x0f
    ly = py - y0f
    hi = wl - 1
    x0 = jnp.clip(x0f.astype(jnp.int32), 0, hi)
    x1i = jnp.clip(x0 + 1, 0, hi)
    y0 = jnp.clip(y0f.astype(jnp.int32), 0, hi)
    y1i = jnp.clip(y0 + 1, 0, hi)
    ys = jnp.where(dy == 0, y0, y1i)
    xs = jnp.where(dx == 0, x0, x1i)
    wy = jnp.where(dy == 0, 1.0 - ly, ly)
    wx = jnp.where(dx == 0, 1.0 - lx, lx)
    idx_ref[...] = basel + ys * wl + xs
    w_ref[...] = wy * wx


def _tc_indices(rois, interpret=False):
    return pl.pallas_call(
        _tc_index_kernel,
        grid=(R // RB,),
        in_specs=[pl.BlockSpec((RB, 5), lambda i: (i, 0))],
        out_specs=[
            pl.BlockSpec((RB, NE), lambda i: (i, 0)),
            pl.BlockSpec((RB, NE), lambda i: (i, 0)),
        ],
        out_shape=[
            jax.ShapeDtypeStruct((R, NE), jnp.int32),
            jax.ShapeDtypeStruct((R, NE), jnp.float32),
        ],
        interpret=interpret,
    )(rois)


@functools.cache
def _sc_gather_fn():
    mesh = plsc.VectorSubcoreMesh(
        core_axis_name="c", subcore_axis_name="s", num_cores=2, num_subcores=16
    )
    return functools.partial(
        pl.kernel,
        out_type=jax.ShapeDtypeStruct((OUT_ROWS_PAD, C), jnp.float32),
        mesh=mesh,
        scratch_types=[
            pltpu.VMEM((2, CH4), jnp.int32),
            pltpu.VMEM((2, CH4), jnp.float32),
            pltpu.VMEM((2, CH4, C), jnp.float32),
            pltpu.VMEM((CH4 // 4, C), jnp.float32),
            pltpu.SemaphoreType.DMA,
            pltpu.SemaphoreType.DMA,
        ],
    )(_sc_gather_body)


def _sc_gather_body(idx_hbm, w_hbm, table_hbm, out_hbm, idx_v, w_v, rows_v, acc_v,
                    gsem0, gsem1):
    wid = lax.axis_index("s") * 2 + lax.axis_index("c")
    base4 = wid * PER_W4
    baser = wid * (PER_W4 // 4)
    sems = (gsem0, gsem1)

    def stage(tt, buf):
        off = base4 + tt * CH4
        pltpu.sync_copy(idx_hbm.at[pl.ds(off, CH4)], idx_v.at[buf])
        pltpu.sync_copy(w_hbm.at[pl.ds(off, CH4)], w_v.at[buf])
        pltpu.async_copy(table_hbm.at[idx_v.at[buf]], rows_v.at[buf], sems[buf])

    def wait(buf):
        pltpu.make_async_copy(
            table_hbm.at[idx_v.at[buf]], rows_v.at[buf], sems[buf]
        ).wait()

    def compute_and_store(tt, buf):
        def body(q, carry):
            wvec = w_v[buf, pl.ds(16 * q, 16)]
            for bb in range(4):
                b = 4 * q + bb
                r = 4 * b
                w0 = wvec[4 * bb]
                w1 = wvec[4 * bb + 1]
                w2 = wvec[4 * bb + 2]
                w3 = wvec[4 * bb + 3]
                for g in range(C // 16):
                    sl = pl.ds(16 * g, 16)
                    acc_v[b, sl] = (
                        w0 * rows_v[buf, r, sl]
                        + w1 * rows_v[buf, r + 1, sl]
                        + w2 * rows_v[buf, r + 2, sl]
                        + w3 * rows_v[buf, r + 3, sl]
                    )
            return carry

        lax.fori_loop(0, CH4 // 16, body, 0)
        pltpu.sync_copy(acc_v, out_hbm.at[pl.ds(baser + tt * (CH4 // 4), CH4 // 4)])

    stage(0, 0)

    def outer(t2, carry):
        tt0 = 2 * t2

        @pl.when(tt0 + 1 < CHUNKS)
        def _():
            stage(tt0 + 1, 1)

        wait(0)
        compute_and_store(tt0, 0)

        @pl.when(tt0 + 2 < CHUNKS)
        def _():
            stage(tt0 + 2, 0)

        @pl.when(tt0 + 1 < CHUNKS)
        def _():
            wait(1)
            compute_and_store(tt0 + 1, 1)

        return carry

    lax.fori_loop(0, (CHUNKS + 1) // 2, outer, 0)


def kernel(feat0, feat1, feat2, feat3, rois):
    idx_all, w_all = _tc_indices(rois)
    pad = M4_PAD - M4
    idx_flat = jnp.concatenate([idx_all.reshape(-1), jnp.zeros((pad,), jnp.int32)])
    w_flat = jnp.concatenate([w_all.reshape(-1), jnp.zeros((pad,), jnp.float32)])
    table = jnp.concatenate(
        [f.reshape(C, -1).T for f in (feat0, feat1, feat2, feat3)], axis=0
    )
    pooled = _sc_gather_fn()(idx_flat, w_flat, table)
    out = pooled[: R * NB].reshape(R, PH, PW, C)
    return jnp.transpose(out, (0, 3, 1, 2))
